# baseline (speedup 1.0000x reference)
import functools

import jax
import jax.numpy as jnp
from jax import lax
from jax.experimental import pallas as pl
from jax.experimental.pallas import tpu as pltpu
from jax.experimental.pallas import tpu_sc as plsc

B = 16384
D = 64
NE = 1000000
NR = 1000
L = 16
NC = 2
NS = 16
NW = NC * NS
BPW = B // NW

BLK = 65536
NBLK = (NE + BLK - 1) // BLK


def _rowsum_body(et_ref, rt_ref, esum_ref, rsum_ref):
    esum_ref[...] = jnp.sum(et_ref[...], axis=0)

    @pl.when(pl.program_id(0) == 0)
    def _():
        rsum_ref[...] = jnp.sum(rt_ref[...], axis=0)


_rowsums = pl.pallas_call(
    _rowsum_body,
    grid=(NBLK,),
    in_specs=[
        pl.BlockSpec((D, BLK), lambda i: (0, i)),
        pl.BlockSpec((D, NR), lambda i: (0, 0)),
    ],
    out_specs=[
        pl.BlockSpec((BLK,), lambda i: (i,)),
        pl.BlockSpec((NR,), lambda i: (0,)),
    ],
    out_shape=[
        jax.ShapeDtypeStruct((NE,), jnp.float32),
        jax.ShapeDtypeStruct((NR,), jnp.float32),
    ],
)


def _build_score():
    mesh = plsc.VectorSubcoreMesh(core_axis_name="c", subcore_axis_name="s")

    cp = pltpu.CompilerParams(
        needs_layout_passes=False,
        use_tc_tiling_on_sc=False,
    )

    @functools.partial(
        pl.kernel,
        mesh=mesh,
        compiler_params=cp,
        out_type=jax.ShapeDtypeStruct((B,), jnp.float32),
        scratch_types=[
            pltpu.VMEM((BPW,), jnp.int32),
            pltpu.VMEM((BPW,), jnp.int32),
            pltpu.VMEM((BPW,), jnp.int32),
            pltpu.VMEM((BPW,), jnp.float32),
            pltpu.VMEM((BPW,), jnp.float32),
            pltpu.VMEM((NR,), jnp.float32),
            pltpu.VMEM((BPW,), jnp.float32),
            pltpu.SemaphoreType.DMA,
        ],
    )
    def score(subj_hbm, rel_hbm, obj_hbm, esum_hbm, rsum_hbm, out_hbm,
              si_v, ri_v, oi_v, es_v, eo_v, rs_v, res_v, sem):
        wid = lax.axis_index("s") * NC + lax.axis_index("c")
        base = wid * BPW

        pltpu.sync_copy(subj_hbm.at[pl.ds(base, BPW)], si_v)
        pltpu.sync_copy(obj_hbm.at[pl.ds(base, BPW)], oi_v)
        pltpu.sync_copy(rel_hbm.at[pl.ds(base, BPW)], ri_v)
        cs = pltpu.async_copy(esum_hbm.at[si_v], es_v, sem)
        co = pltpu.async_copy(esum_hbm.at[oi_v], eo_v, sem)
        cr = pltpu.async_copy(rsum_hbm, rs_v, sem)
        cs.wait()
        co.wait()
        cr.wait()

        @pl.loop(0, BPW // L)
        def _(c):
            sl = pl.ds(c * L, L)
            rel_idx = ri_v[sl]
            r = plsc.load_gather(rs_v, [rel_idx])
            res_v[sl] = es_v[sl] + r - eo_v[sl]

        pltpu.sync_copy(res_v, out_hbm.at[pl.ds(base, BPW)])

    return score


_score = _build_score()


@jax.jit
def kernel(subject, relation, object, embed_entities, embed_relations):
    esum, rsum = _rowsums(embed_entities.T, embed_relations.T)
    out = _score(
        subject.astype(jnp.int32),
        relation.astype(jnp.int32),
        object.astype(jnp.int32),
        esum,
        rsum,
    )
    return out.reshape(-1, 1)


# speedup vs baseline: 1.0040x; 1.0040x over previous
import functools

import jax
import jax.numpy as jnp
from jax import lax
from jax.experimental import pallas as pl
from jax.experimental.pallas import tpu as pltpu
from jax.experimental.pallas import tpu_sc as plsc

B = 16384
D = 64
NE = 1000000
NR = 1000
L = 16
NC = 2
NS = 16
NW = NC * NS
BPW = B // NW

BLK = 32768
NBLK = (NE + BLK - 1) // BLK


def _rowsum_body(et_ref, rt_ref, esum_ref, rsum_ref):
    esum_ref[...] = jnp.sum(et_ref[...], axis=0)

    @pl.when(pl.program_id(0) == 0)
    def _():
        rsum_ref[...] = jnp.sum(rt_ref[...], axis=0)


_rowsums = pl.pallas_call(
    _rowsum_body,
    grid=(NBLK,),
    in_specs=[
        pl.BlockSpec((D, BLK), lambda i: (0, i)),
        pl.BlockSpec((D, NR), lambda i: (0, 0)),
    ],
    out_specs=[
        pl.BlockSpec((BLK,), lambda i: (i,)),
        pl.BlockSpec((NR,), lambda i: (0,)),
    ],
    out_shape=[
        jax.ShapeDtypeStruct((NE,), jnp.float32),
        jax.ShapeDtypeStruct((NR,), jnp.float32),
    ],
    compiler_params=pltpu.CompilerParams(
        dimension_semantics=("parallel",),
    ),
)


def _build_score():
    mesh = plsc.VectorSubcoreMesh(core_axis_name="c", subcore_axis_name="s")

    cp = pltpu.CompilerParams(
        needs_layout_passes=False,
        use_tc_tiling_on_sc=False,
    )

    @functools.partial(
        pl.kernel,
        mesh=mesh,
        compiler_params=cp,
        out_type=jax.ShapeDtypeStruct((B,), jnp.float32),
        scratch_types=[
            pltpu.VMEM((BPW,), jnp.int32),
            pltpu.VMEM((BPW,), jnp.int32),
            pltpu.VMEM((BPW,), jnp.int32),
            pltpu.VMEM((BPW,), jnp.float32),
            pltpu.VMEM((BPW,), jnp.float32),
            pltpu.VMEM((NR,), jnp.float32),
            pltpu.VMEM((BPW,), jnp.float32),
            pltpu.SemaphoreType.DMA,
        ],
    )
    def score(subj_hbm, rel_hbm, obj_hbm, esum_hbm, rsum_hbm, out_hbm,
              si_v, ri_v, oi_v, es_v, eo_v, rs_v, res_v, sem):
        wid = lax.axis_index("s") * NC + lax.axis_index("c")
        base = wid * BPW

        pltpu.sync_copy(subj_hbm.at[pl.ds(base, BPW)], si_v)
        pltpu.sync_copy(obj_hbm.at[pl.ds(base, BPW)], oi_v)
        pltpu.sync_copy(rel_hbm.at[pl.ds(base, BPW)], ri_v)
        cs = pltpu.async_copy(esum_hbm.at[si_v], es_v, sem)
        co = pltpu.async_copy(esum_hbm.at[oi_v], eo_v, sem)
        cr = pltpu.async_copy(rsum_hbm, rs_v, sem)
        cs.wait()
        co.wait()
        cr.wait()

        @pl.loop(0, BPW // L)
        def _(c):
            sl = pl.ds(c * L, L)
            rel_idx = ri_v[sl]
            r = plsc.load_gather(rs_v, [rel_idx])
            res_v[sl] = es_v[sl] + r - eo_v[sl]

        pltpu.sync_copy(res_v, out_hbm.at[pl.ds(base, BPW)])

    return score


_score = _build_score()


@jax.jit
def kernel(subject, relation, object, embed_entities, embed_relations):
    esum, rsum = _rowsums(embed_entities.T, embed_relations.T)
    out = _score(
        subject.astype(jnp.int32),
        relation.astype(jnp.int32),
        object.astype(jnp.int32),
        esum,
        rsum,
    )
    return out.reshape(-1, 1)


# speedup vs baseline: 1.0075x; 1.0035x over previous
import functools

import jax
import jax.numpy as jnp
from jax import lax
from jax.experimental import pallas as pl
from jax.experimental.pallas import tpu as pltpu
from jax.experimental.pallas import tpu_sc as plsc

B = 16384
D = 64
NE = 1000000
NR = 1000
L = 16
NC = 2
NS = 16
NW = NC * NS
BPW = B // NW

BLK = 32768
NBLK = (NE + BLK - 1) // BLK


def _rowsum_body(et_ref, rt_ref, esum_ref, rsum_ref):
    esum_ref[...] = jnp.sum(et_ref[...], axis=0)

    @pl.when(pl.program_id(0) == 0)
    def _():
        rsum_ref[...] = jnp.sum(rt_ref[...], axis=0)


_rowsums = pl.pallas_call(
    _rowsum_body,
    grid=(NBLK,),
    in_specs=[
        pl.BlockSpec((D, BLK), lambda i: (0, i)),
        pl.BlockSpec((D, NR), lambda i: (0, 0)),
    ],
    out_specs=[
        pl.BlockSpec((BLK,), lambda i: (i,)),
        pl.BlockSpec((NR,), lambda i: (0,)),
    ],
    out_shape=[
        jax.ShapeDtypeStruct((NE,), jnp.float32),
        jax.ShapeDtypeStruct((NR,), jnp.float32),
    ],
)


def _build_score():
    mesh = plsc.VectorSubcoreMesh(core_axis_name="c", subcore_axis_name="s")

    cp = pltpu.CompilerParams(
        needs_layout_passes=False,
        use_tc_tiling_on_sc=False,
    )

    @functools.partial(
        pl.kernel,
        mesh=mesh,
        compiler_params=cp,
        out_type=jax.ShapeDtypeStruct((B,), jnp.float32),
        scratch_types=[
            pltpu.VMEM((BPW,), jnp.int32),
            pltpu.VMEM((BPW,), jnp.int32),
            pltpu.VMEM((BPW,), jnp.int32),
            pltpu.VMEM((BPW,), jnp.float32),
            pltpu.VMEM((BPW,), jnp.float32),
            pltpu.VMEM((NR,), jnp.float32),
            pltpu.VMEM((BPW,), jnp.float32),
            pltpu.SemaphoreType.DMA,
        ],
    )
    def score(subj_hbm, rel_hbm, obj_hbm, esum_hbm, rsum_hbm, out_hbm,
              si_v, ri_v, oi_v, es_v, eo_v, rs_v, res_v, sem):
        wid = lax.axis_index("s") * NC + lax.axis_index("c")
        base = wid * BPW

        pltpu.sync_copy(subj_hbm.at[pl.ds(base, BPW)], si_v)
        pltpu.sync_copy(obj_hbm.at[pl.ds(base, BPW)], oi_v)
        pltpu.sync_copy(rel_hbm.at[pl.ds(base, BPW)], ri_v)
        cs = pltpu.async_copy(esum_hbm.at[si_v], es_v, sem)
        co = pltpu.async_copy(esum_hbm.at[oi_v], eo_v, sem)
        cr = pltpu.async_copy(rsum_hbm, rs_v, sem)
        cs.wait()
        co.wait()
        cr.wait()

        @pl.loop(0, BPW // L)
        def _(c):
            sl = pl.ds(c * L, L)
            rel_idx = ri_v[sl]
            r = plsc.load_gather(rs_v, [rel_idx])
            res_v[sl] = es_v[sl] + r - eo_v[sl]

        pltpu.sync_copy(res_v, out_hbm.at[pl.ds(base, BPW)])

    return score


_score = _build_score()


@jax.jit
def kernel(subject, relation, object, embed_entities, embed_relations):
    esum, rsum = _rowsums(embed_entities.T, embed_relations.T)
    out = _score(
        subject.astype(jnp.int32),
        relation.astype(jnp.int32),
        object.astype(jnp.int32),
        esum,
        rsum,
    )
    return out.reshape(-1, 1)


# speedup vs baseline: 1.0208x; 1.0132x over previous
import functools

import jax
import jax.numpy as jnp
from jax import lax
from jax.experimental import pallas as pl
from jax.experimental.pallas import tpu as pltpu
from jax.experimental.pallas import tpu_sc as plsc

B = 16384
D = 64
NE = 1000000
NR = 1000
L = 16
NC = 2
NS = 16
NW = NC * NS
BPW = B // NW

BLK = 16384
NSTREAM = 2
STEP = NSTREAM * BLK
NSTEP = (NE + STEP - 1) // STEP


def _rowsum_body(*refs):
    et_refs, rt_ref = refs[:NSTREAM], refs[NSTREAM]
    esum_ref, rsum_ref = refs[NSTREAM + 1], refs[NSTREAM + 2]
    for q in range(NSTREAM):
        esum_ref[pl.ds(q * BLK, BLK)] = jnp.sum(et_refs[q][...], axis=0)

    @pl.when(pl.program_id(0) == 0)
    def _():
        rsum_ref[...] = jnp.sum(rt_ref[...], axis=0)


_rowsums = pl.pallas_call(
    _rowsum_body,
    grid=(NSTEP,),
    in_specs=[
        pl.BlockSpec((D, BLK), (lambda i, q=q: (0, NSTREAM * i + q)))
        for q in range(NSTREAM)
    ] + [
        pl.BlockSpec((D, NR), lambda i: (0, 0)),
    ],
    out_specs=[
        pl.BlockSpec((STEP,), lambda i: (i,)),
        pl.BlockSpec((NR,), lambda i: (0,)),
    ],
    out_shape=[
        jax.ShapeDtypeStruct((NE,), jnp.float32),
        jax.ShapeDtypeStruct((NR,), jnp.float32),
    ],
)


def _build_score():
    mesh = plsc.VectorSubcoreMesh(core_axis_name="c", subcore_axis_name="s")

    cp = pltpu.CompilerParams(
        needs_layout_passes=False,
        use_tc_tiling_on_sc=False,
    )

    @functools.partial(
        pl.kernel,
        mesh=mesh,
        compiler_params=cp,
        out_type=jax.ShapeDtypeStruct((B,), jnp.float32),
        scratch_types=[
            pltpu.VMEM((BPW,), jnp.int32),
            pltpu.VMEM((BPW,), jnp.int32),
            pltpu.VMEM((BPW,), jnp.int32),
            pltpu.VMEM((BPW,), jnp.float32),
            pltpu.VMEM((BPW,), jnp.float32),
            pltpu.VMEM((NR,), jnp.float32),
            pltpu.VMEM((BPW,), jnp.float32),
            pltpu.SemaphoreType.DMA,
        ],
    )
    def score(subj_hbm, rel_hbm, obj_hbm, esum_hbm, rsum_hbm, out_hbm,
              si_v, ri_v, oi_v, es_v, eo_v, rs_v, res_v, sem):
        wid = lax.axis_index("s") * NC + lax.axis_index("c")
        base = wid * BPW

        pltpu.sync_copy(subj_hbm.at[pl.ds(base, BPW)], si_v)
        pltpu.sync_copy(obj_hbm.at[pl.ds(base, BPW)], oi_v)
        pltpu.sync_copy(rel_hbm.at[pl.ds(base, BPW)], ri_v)
        cs = pltpu.async_copy(esum_hbm.at[si_v], es_v, sem)
        co = pltpu.async_copy(esum_hbm.at[oi_v], eo_v, sem)
        cr = pltpu.async_copy(rsum_hbm, rs_v, sem)
        cs.wait()
        co.wait()
        cr.wait()

        @pl.loop(0, BPW // L)
        def _(c):
            sl = pl.ds(c * L, L)
            rel_idx = ri_v[sl]
            r = plsc.load_gather(rs_v, [rel_idx])
            res_v[sl] = es_v[sl] + r - eo_v[sl]

        pltpu.sync_copy(res_v, out_hbm.at[pl.ds(base, BPW)])

    return score


_score = _build_score()


@jax.jit
def kernel(subject, relation, object, embed_entities, embed_relations):
    et = embed_entities.T
    esum, rsum = _rowsums(*([et] * NSTREAM), embed_relations.T)
    out = _score(
        subject.astype(jnp.int32),
        relation.astype(jnp.int32),
        object.astype(jnp.int32),
        esum,
        rsum,
    )
    return out.reshape(-1, 1)
